# baseline (speedup 1.0000x reference)
import dataclasses
import functools

import jax
import jax.numpy as jnp
from jax import lax
from jax.experimental import pallas as pl
from jax.experimental.pallas import tpu as pltpu
from jax.experimental.pallas import tpu_sc as plsc

HIDDEN = 1024
FFN = 2048
NE = 8
T = 2048
TOPK = 2
TM = 256
F_TILE = 512
NF = FFN // F_TILE
NPAD = T * TOPK + NE * TM
NT = NPAD // TM
NW = 32
TPW = T // NW
CH = 256


def _router_body(x_ref, gw_ref, logits_ref, w0_ref, w1_ref,
                 p0_ref, p1_ref, te_ref, e0_ref, e1_ref, r0_ref, r1_ref,
                 h_ref):
    x = x_ref[...]
    logits = lax.dot_general(x, gw_ref[...], (((1,), (1,)), ((), ())),
                             preferred_element_type=jnp.float32)
    logits_ref[...] = logits
    col = lax.broadcasted_iota(jnp.int32, logits.shape, 1)
    m1 = jnp.max(logits, axis=1, keepdims=True)
    e0 = jnp.min(jnp.where(logits == m1, col, NE), axis=1, keepdims=True)
    masked = jnp.where(col == e0, jnp.float32(-1e30), logits)
    m2 = jnp.max(masked, axis=1, keepdims=True)
    e1 = jnp.min(jnp.where(masked == m2, col, NE), axis=1, keepdims=True)
    s = jnp.exp(m2 - m1)
    denom = 1.0 + s
    e0_ref[...] = e0
    e1_ref[...] = e1
    w0_ref[...] = 1.0 / denom
    w1_ref[...] = s / denom
    h_ref[...] = ((col == e0) | (col == e1)).astype(jnp.float32)

    ri = lax.broadcasted_iota(jnp.int32, (CH, CH), 0)
    ci = lax.broadcasted_iota(jnp.int32, (CH, CH), 1)
    tri = (ri > ci).astype(jnp.bfloat16)

    def chunk(i, carry):
        sl = pl.ds(i * CH, CH)
        hc = h_ref[sl, :]
        cc = lax.dot_general(tri, hc.astype(jnp.bfloat16),
                             (((1,), (0,)), ((), ())),
                             preferred_element_type=jnp.float32) + carry
        colc = lax.broadcasted_iota(jnp.int32, (CH, NE), 1)
        e0c = e0_ref[sl, :]
        e1c = e1_ref[sl, :]
        zero = jnp.float32(0.0)
        r0_ref[sl, :] = jnp.sum(jnp.where(colc == e0c, cc, zero), axis=1,
                                keepdims=True)
        r1_ref[sl, :] = jnp.sum(jnp.where(colc == e1c, cc, zero), axis=1,
                                keepdims=True)
        return carry + jnp.sum(hc, axis=0, keepdims=True)

    cnt = lax.fori_loop(0, T // CH, chunk, jnp.zeros((1, NE), jnp.float32))

    tmf = jnp.float32(TM)
    padded = jnp.floor((cnt + (TM - 1)) / tmf) * tmf
    er = lax.broadcasted_iota(jnp.int32, (NE, NE), 0)
    ec = lax.broadcasted_iota(jnp.int32, (NE, NE), 1)
    t8 = (er < ec).astype(jnp.float32)
    offs = lax.dot_general(padded, t8, (((1,), (0,)), ((), ())),
                           preferred_element_type=jnp.float32)
    off0 = jnp.sum(jnp.where(col == e0, offs, jnp.float32(0.0)), axis=1,
                   keepdims=True)
    off1 = jnp.sum(jnp.where(col == e1, offs, jnp.float32(0.0)), axis=1,
                   keepdims=True)
    p0_ref[...] = (off0 + r0_ref[...]).astype(jnp.int32)
    p1_ref[...] = (off1 + r1_ref[...]).astype(jnp.int32)
    starts = offs / tmf
    tj = lax.broadcasted_iota(jnp.int32, (NT, NE), 0).astype(jnp.float32)
    te_ref[...] = (jnp.sum((tj >= starts).astype(jnp.float32), axis=1,
                           keepdims=True) - 1.0).astype(jnp.int32)


def _grouped_body(te_ref, xs_ref, w1_ref, w3_ref, w2_ref, out_ref,
                  xsb_ref, w1b_ref, w3b_ref, w2b_ref):
    f = pl.program_id(0)
    i = pl.program_id(1)
    sl = pl.ds(i * TM, TM)
    prev = te_ref[jnp.maximum(i - 1, 0), 0]
    changed = (i == 0) | (te_ref[i, 0] != prev)

    @pl.when(changed)
    def _cast():
        w1b_ref[...] = w1_ref[0].astype(jnp.bfloat16)
        w3b_ref[...] = w3_ref[0].astype(jnp.bfloat16)
        w2b_ref[...] = w2_ref[0].astype(jnp.bfloat16)

    @pl.when(f == 0)
    def _cx():
        xsb_ref[sl, :] = xs_ref[...].astype(jnp.bfloat16)

    xb = xsb_ref[sl, :]
    y1 = lax.dot_general(xb, w1b_ref[...], (((1,), (1,)), ((), ())),
                         preferred_element_type=jnp.float32)
    y3 = lax.dot_general(xb, w3b_ref[...], (((1,), (1,)), ((), ())),
                         preferred_element_type=jnp.float32)
    h = ((y1 * lax.logistic(y1)) * y3).astype(jnp.bfloat16)
    yp = lax.dot_general(h, w2b_ref[...], (((1,), (1,)), ((), ())),
                         preferred_element_type=jnp.float32)

    @pl.when(f == 0)
    def _set():
        out_ref[sl, :] = yp

    @pl.when(f != 0)
    def _acc():
        out_ref[sl, :] += yp


def _sc_mesh():
    return plsc.VectorSubcoreMesh(core_axis_name="c", subcore_axis_name="s")


def _dispatch_scatter(x_f32, pos0, pos1):
    @functools.partial(
        pl.kernel, mesh=_sc_mesh(),
        out_type=jax.ShapeDtypeStruct((NPAD, HIDDEN), jnp.float32),
        scratch_types=[
            pltpu.VMEM((TPW,), jnp.int32),
            pltpu.VMEM((TPW,), jnp.int32),
            pltpu.VMEM((TPW, HIDDEN), jnp.float32),
            pltpu.SemaphoreType.DMA,
        ],
    )
    def k(x_hbm, p0_hbm, p1_hbm, xs_hbm, i0_v, i1_v, rows_v, sem):
        wid = lax.axis_index("s") * 2 + lax.axis_index("c")
        base = wid * TPW
        pltpu.sync_copy(p0_hbm.at[pl.ds(base, TPW)], i0_v)
        pltpu.sync_copy(p1_hbm.at[pl.ds(base, TPW)], i1_v)
        pltpu.sync_copy(x_hbm.at[pl.ds(base, TPW)], rows_v)
        pltpu.async_copy(rows_v, xs_hbm.at[i0_v], sem).wait()
        pltpu.async_copy(rows_v, xs_hbm.at[i1_v], sem).wait()

    return k(x_f32, pos0, pos1)


HH = TPW // 2


def _combine_fused(ys, pos0, pos1, wt0, wt1):
    cp = pltpu.CompilerParams()
    if "needs_layout_passes" in pltpu.CompilerParams.__dataclass_fields__:
        cp = dataclasses.replace(cp, needs_layout_passes=False)

    @functools.partial(
        pl.kernel, mesh=_sc_mesh(), compiler_params=cp,
        out_type=jax.ShapeDtypeStruct((T, HIDDEN), jnp.float32),
        scratch_types=[
            pltpu.VMEM((TPW,), jnp.int32),
            pltpu.VMEM((TPW,), jnp.int32),
            pltpu.VMEM((TPW,), jnp.float32),
            pltpu.VMEM((TPW,), jnp.float32),
            pltpu.VMEM((HH, HIDDEN), jnp.float32),
            pltpu.VMEM((HH, HIDDEN), jnp.float32),
            pltpu.SemaphoreType.DMA,
        ],
    )
    def k(ys_hbm, p0_hbm, p1_hbm, wt0_hbm, wt1_hbm, out_hbm,
          i0_v, i1_v, w0_v, w1_v, a_v, b_v, sem):
        wid = lax.axis_index("s") * 2 + lax.axis_index("c")
        base = wid * TPW
        pltpu.sync_copy(p0_hbm.at[pl.ds(base, TPW)], i0_v)
        pltpu.sync_copy(p1_hbm.at[pl.ds(base, TPW)], i1_v)
        pltpu.sync_copy(wt0_hbm.at[pl.ds(base, TPW)], w0_v)
        pltpu.sync_copy(wt1_hbm.at[pl.ds(base, TPW)], w1_v)
        for hb in (0, HH):
            pltpu.async_copy(ys_hbm.at[i0_v.at[pl.ds(hb, HH)]], a_v,
                             sem).wait()
            pltpu.async_copy(ys_hbm.at[i1_v.at[pl.ds(hb, HH)]], b_v,
                             sem).wait()

            @pl.loop(0, HH)
            def _row(r):
                idx = jnp.full((16,), hb + r, jnp.int32)
                w0s = plsc.load_gather(w0_v, [idx])
                w1s = plsc.load_gather(w1_v, [idx])

                @pl.loop(0, HIDDEN // 16)
                def _chunk(c):
                    cs = pl.ds(c * 16, 16)
                    a_v[r, cs] = a_v[r, cs] * w0s + b_v[r, cs] * w1s

            pltpu.sync_copy(a_v, out_hbm.at[pl.ds(base + hb, HH)])

    return k(ys, pos0, pos1, wt0, wt1)


def kernel(hidden_states, gate_w, w1, w3, w2):
    b, s, hd = hidden_states.shape
    x2 = hidden_states.reshape(T, HIDDEN)

    (logits, wt0, wt1, pos0, pos1, tile_expert) = pl.pallas_call(
        _router_body,
        out_shape=(
            jax.ShapeDtypeStruct((T, NE), jnp.float32),
            jax.ShapeDtypeStruct((T, 1), jnp.float32),
            jax.ShapeDtypeStruct((T, 1), jnp.float32),
            jax.ShapeDtypeStruct((T, 1), jnp.int32),
            jax.ShapeDtypeStruct((T, 1), jnp.int32),
            jax.ShapeDtypeStruct((NT, 1), jnp.int32),
        ),
        scratch_shapes=[
            pltpu.VMEM((T, 1), jnp.int32),
            pltpu.VMEM((T, 1), jnp.int32),
            pltpu.VMEM((T, 1), jnp.float32),
            pltpu.VMEM((T, 1), jnp.float32),
            pltpu.VMEM((T, NE), jnp.float32),
        ],
    )(x2, gate_w)
    pos0 = pos0.reshape(T)
    pos1 = pos1.reshape(T)

    xs = _dispatch_scatter(x2, pos0, pos1)

    ys = pl.pallas_call(
        _grouped_body,
        grid_spec=pltpu.PrefetchScalarGridSpec(
            num_scalar_prefetch=1,
            grid=(NF, NT),
            in_specs=[
                pl.BlockSpec((TM, HIDDEN),
                             lambda f, i, te: (jnp.where(f == 0, i, 0), 0)),
                pl.BlockSpec((1, F_TILE, HIDDEN), lambda f, i, te: (te[i, 0], f, 0)),
                pl.BlockSpec((1, F_TILE, HIDDEN), lambda f, i, te: (te[i, 0], f, 0)),
                pl.BlockSpec((1, HIDDEN, F_TILE), lambda f, i, te: (te[i, 0], 0, f)),
            ],
            out_specs=pl.BlockSpec((NPAD, HIDDEN), lambda f, i, te: (0, 0)),
            scratch_shapes=[
                pltpu.VMEM((NPAD, HIDDEN), jnp.bfloat16),
                pltpu.VMEM((F_TILE, HIDDEN), jnp.bfloat16),
                pltpu.VMEM((F_TILE, HIDDEN), jnp.bfloat16),
                pltpu.VMEM((HIDDEN, F_TILE), jnp.bfloat16),
            ],
        ),
        out_shape=jax.ShapeDtypeStruct((NPAD, HIDDEN), jnp.float32),
    )(tile_expert, xs, w1, w3, w2)

    out = _combine_fused(ys, pos0, pos1, wt0.reshape(T), wt1.reshape(T))

    return out.reshape(b, s, hd), logits


# speedup vs baseline: 1.0483x; 1.0483x over previous
import functools

import jax
import jax.numpy as jnp
from jax import lax
from jax.experimental import pallas as pl
from jax.experimental.pallas import tpu as pltpu
from jax.experimental.pallas import tpu_sc as plsc

HIDDEN = 1024
FFN = 2048
NE = 8
T = 2048
TOPK = 2
TM = 256
F_TILE = 512
NF = FFN // F_TILE
NPAD = T * TOPK + NE * TM
NT = NPAD // TM
NW = 32
TPW = T // NW
CH = 256


def _router_body(x_ref, gw_ref, logits_ref, w0_ref, w1_ref,
                 p0_ref, p1_ref, te_ref, e0_ref, e1_ref, r0_ref, r1_ref,
                 h_ref):
    x = x_ref[...]
    logits = lax.dot_general(x, gw_ref[...], (((1,), (1,)), ((), ())),
                             preferred_element_type=jnp.float32)
    logits_ref[...] = logits
    col = lax.broadcasted_iota(jnp.int32, logits.shape, 1)
    m1 = jnp.max(logits, axis=1, keepdims=True)
    e0 = jnp.min(jnp.where(logits == m1, col, NE), axis=1, keepdims=True)
    masked = jnp.where(col == e0, jnp.float32(-1e30), logits)
    m2 = jnp.max(masked, axis=1, keepdims=True)
    e1 = jnp.min(jnp.where(masked == m2, col, NE), axis=1, keepdims=True)
    s = jnp.exp(m2 - m1)
    denom = 1.0 + s
    e0_ref[...] = e0
    e1_ref[...] = e1
    w0_ref[...] = 1.0 / denom
    w1_ref[...] = s / denom
    h_ref[...] = ((col == e0) | (col == e1)).astype(jnp.float32)

    ri = lax.broadcasted_iota(jnp.int32, (CH, CH), 0)
    ci = lax.broadcasted_iota(jnp.int32, (CH, CH), 1)
    tri = (ri > ci).astype(jnp.bfloat16)

    def chunk(i, carry):
        sl = pl.ds(i * CH, CH)
        hc = h_ref[sl, :]
        cc = lax.dot_general(tri, hc.astype(jnp.bfloat16),
                             (((1,), (0,)), ((), ())),
                             preferred_element_type=jnp.float32) + carry
        colc = lax.broadcasted_iota(jnp.int32, (CH, NE), 1)
        e0c = e0_ref[sl, :]
        e1c = e1_ref[sl, :]
        zero = jnp.float32(0.0)
        r0_ref[sl, :] = jnp.sum(jnp.where(colc == e0c, cc, zero), axis=1,
                                keepdims=True)
        r1_ref[sl, :] = jnp.sum(jnp.where(colc == e1c, cc, zero), axis=1,
                                keepdims=True)
        return carry + jnp.sum(hc, axis=0, keepdims=True)

    cnt = lax.fori_loop(0, T // CH, chunk, jnp.zeros((1, NE), jnp.float32))

    tmf = jnp.float32(TM)
    padded = jnp.floor((cnt + (TM - 1)) / tmf) * tmf
    er = lax.broadcasted_iota(jnp.int32, (NE, NE), 0)
    ec = lax.broadcasted_iota(jnp.int32, (NE, NE), 1)
    t8 = (er < ec).astype(jnp.float32)
    offs = lax.dot_general(padded, t8, (((1,), (0,)), ((), ())),
                           preferred_element_type=jnp.float32)
    off0 = jnp.sum(jnp.where(col == e0, offs, jnp.float32(0.0)), axis=1,
                   keepdims=True)
    off1 = jnp.sum(jnp.where(col == e1, offs, jnp.float32(0.0)), axis=1,
                   keepdims=True)
    p0_ref[...] = (off0 + r0_ref[...]).astype(jnp.int32)
    p1_ref[...] = (off1 + r1_ref[...]).astype(jnp.int32)
    starts = offs / tmf
    tj = lax.broadcasted_iota(jnp.int32, (NT, NE), 0).astype(jnp.float32)
    te_ref[...] = (jnp.sum((tj >= starts).astype(jnp.float32), axis=1,
                           keepdims=True) - 1.0).astype(jnp.int32)


def _grouped_body(te_ref, xs_ref, w1_ref, w3_ref, w2_ref, out_ref,
                  xsb_ref, w1b_ref, w3b_ref, w2b_ref):
    f = pl.program_id(0)
    i = pl.program_id(1)
    sl = pl.ds(i * TM, TM)
    prev = te_ref[jnp.maximum(i - 1, 0), 0]
    changed = (i == 0) | (te_ref[i, 0] != prev)

    @pl.when(changed)
    def _cast():
        w1b_ref[...] = w1_ref[0].astype(jnp.bfloat16)
        w3b_ref[...] = w3_ref[0].astype(jnp.bfloat16)
        w2b_ref[...] = w2_ref[0].astype(jnp.bfloat16)

    @pl.when(f == 0)
    def _cx():
        xsb_ref[sl, :] = xs_ref[...].astype(jnp.bfloat16)

    xb = xsb_ref[sl, :]
    y1 = lax.dot_general(xb, w1b_ref[...], (((1,), (1,)), ((), ())),
                         preferred_element_type=jnp.float32)
    y3 = lax.dot_general(xb, w3b_ref[...], (((1,), (1,)), ((), ())),
                         preferred_element_type=jnp.float32)
    h = ((y1 * lax.logistic(y1)) * y3).astype(jnp.bfloat16)
    yp = lax.dot_general(h, w2b_ref[...], (((1,), (1,)), ((), ())),
                         preferred_element_type=jnp.float32)

    @pl.when(f == 0)
    def _set():
        out_ref[sl, :] = yp

    @pl.when(f != 0)
    def _acc():
        out_ref[sl, :] += yp


def _combine_body(a_ref, b_ref, w0_ref, w1_ref, o_ref):
    o_ref[...] = a_ref[...] * w0_ref[...] + b_ref[...] * w1_ref[...]


def _sc_mesh():
    return plsc.VectorSubcoreMesh(core_axis_name="c", subcore_axis_name="s")


def _dispatch_scatter(x_f32, pos0, pos1):
    @functools.partial(
        pl.kernel, mesh=_sc_mesh(),
        out_type=jax.ShapeDtypeStruct((NPAD, HIDDEN), jnp.float32),
        scratch_types=[
            pltpu.VMEM((TPW,), jnp.int32),
            pltpu.VMEM((TPW,), jnp.int32),
            pltpu.VMEM((TPW, HIDDEN), jnp.float32),
            pltpu.SemaphoreType.DMA,
        ],
    )
    def k(x_hbm, p0_hbm, p1_hbm, xs_hbm, i0_v, i1_v, rows_v, sem):
        wid = lax.axis_index("s") * 2 + lax.axis_index("c")
        base = wid * TPW
        pltpu.sync_copy(p0_hbm.at[pl.ds(base, TPW)], i0_v)
        pltpu.sync_copy(p1_hbm.at[pl.ds(base, TPW)], i1_v)
        pltpu.sync_copy(x_hbm.at[pl.ds(base, TPW)], rows_v)
        pltpu.async_copy(rows_v, xs_hbm.at[i0_v], sem).wait()
        pltpu.async_copy(rows_v, xs_hbm.at[i1_v], sem).wait()

    return k(x_f32, pos0, pos1)


def _combine_gather(ys, pos0, pos1):
    @functools.partial(
        pl.kernel, mesh=_sc_mesh(),
        out_type=(jax.ShapeDtypeStruct((T, HIDDEN), jnp.float32),
                  jax.ShapeDtypeStruct((T, HIDDEN), jnp.float32)),
        scratch_types=[
            pltpu.VMEM((TPW,), jnp.int32),
            pltpu.VMEM((TPW,), jnp.int32),
            pltpu.VMEM((TPW, HIDDEN), jnp.float32),
            pltpu.SemaphoreType.DMA,
        ],
    )
    def k(ys_hbm, p0_hbm, p1_hbm, a_hbm, b_hbm, i0_v, i1_v, rows_v, sem):
        wid = lax.axis_index("s") * 2 + lax.axis_index("c")
        base = wid * TPW
        pltpu.sync_copy(p0_hbm.at[pl.ds(base, TPW)], i0_v)
        pltpu.sync_copy(p1_hbm.at[pl.ds(base, TPW)], i1_v)
        pltpu.async_copy(ys_hbm.at[i0_v], rows_v, sem).wait()
        pltpu.sync_copy(rows_v, a_hbm.at[pl.ds(base, TPW)])
        pltpu.async_copy(ys_hbm.at[i1_v], rows_v, sem).wait()
        pltpu.sync_copy(rows_v, b_hbm.at[pl.ds(base, TPW)])

    return k(ys, pos0, pos1)


def kernel(hidden_states, gate_w, w1, w3, w2):
    b, s, hd = hidden_states.shape
    x2 = hidden_states.reshape(T, HIDDEN)

    (logits, wt0, wt1, pos0, pos1, tile_expert) = pl.pallas_call(
        _router_body,
        out_shape=(
            jax.ShapeDtypeStruct((T, NE), jnp.float32),
            jax.ShapeDtypeStruct((T, 1), jnp.float32),
            jax.ShapeDtypeStruct((T, 1), jnp.float32),
            jax.ShapeDtypeStruct((T, 1), jnp.int32),
            jax.ShapeDtypeStruct((T, 1), jnp.int32),
            jax.ShapeDtypeStruct((NT, 1), jnp.int32),
        ),
        scratch_shapes=[
            pltpu.VMEM((T, 1), jnp.int32),
            pltpu.VMEM((T, 1), jnp.int32),
            pltpu.VMEM((T, 1), jnp.float32),
            pltpu.VMEM((T, 1), jnp.float32),
            pltpu.VMEM((T, NE), jnp.float32),
        ],
    )(x2, gate_w)
    pos0 = pos0.reshape(T)
    pos1 = pos1.reshape(T)

    xs = _dispatch_scatter(x2, pos0, pos1)

    ys = pl.pallas_call(
        _grouped_body,
        grid_spec=pltpu.PrefetchScalarGridSpec(
            num_scalar_prefetch=1,
            grid=(NF, NT),
            in_specs=[
                pl.BlockSpec((TM, HIDDEN),
                             lambda f, i, te: (jnp.where(f == 0, i, 0), 0)),
                pl.BlockSpec((1, F_TILE, HIDDEN), lambda f, i, te: (te[i, 0], f, 0)),
                pl.BlockSpec((1, F_TILE, HIDDEN), lambda f, i, te: (te[i, 0], f, 0)),
                pl.BlockSpec((1, HIDDEN, F_TILE), lambda f, i, te: (te[i, 0], 0, f)),
            ],
            out_specs=pl.BlockSpec((NPAD, HIDDEN), lambda f, i, te: (0, 0)),
            scratch_shapes=[
                pltpu.VMEM((NPAD, HIDDEN), jnp.bfloat16),
                pltpu.VMEM((F_TILE, HIDDEN), jnp.bfloat16),
                pltpu.VMEM((F_TILE, HIDDEN), jnp.bfloat16),
                pltpu.VMEM((HIDDEN, F_TILE), jnp.bfloat16),
            ],
        ),
        out_shape=jax.ShapeDtypeStruct((NPAD, HIDDEN), jnp.float32),
    )(tile_expert, xs, w1, w3, w2)

    a, bb = _combine_gather(ys, pos0, pos1)

    out = pl.pallas_call(
        _combine_body,
        out_shape=jax.ShapeDtypeStruct((T, HIDDEN), jnp.float32),
    )(a, bb, wt0, wt1)

    return out.reshape(b, s, hd), logits
